# baseline (speedup 1.0000x reference)
import functools

import jax
import jax.numpy as jnp
from jax import lax
from jax.experimental import pallas as pl
from jax.experimental.pallas import tpu as pltpu
from jax.experimental.pallas import tpu_sc as plsc

N_NODES = 10000
N_EDGES = 160000
DIM = 128
BLK = 4

NC = 2
NS = 16
NW = NC * NS
LANES = 16

N_PAD = 10240
DUMMY_DST = N_PAD - 1
CNT_ROWS = N_PAD // DIM
CHUNK = 16
NCHUNK = 316
EW = CHUNK * NCHUNK
E_PAD = EW * NW
ROWS_PER_TILE = N_PAD // NS
CNT_TILES = 10


def _sc_aggregate(h, wt, src, rel, dst):
  mesh = plsc.VectorSubcoreMesh(
      core_axis_name="c", subcore_axis_name="s", num_cores=NC, num_subcores=NS
  )

  idx_t = pltpu.VMEM((CHUNK,), jnp.int32)
  row_t = pltpu.VMEM((CHUNK, DIM), jnp.float32)
  wrow_t = pltpu.VMEM((CHUNK, 4 * DIM), jnp.float32)

  @functools.partial(
      pl.kernel,
      out_type=(
          jax.ShapeDtypeStruct((NC, N_PAD, DIM), jnp.float32),
          jax.ShapeDtypeStruct((NC, CNT_ROWS, DIM), jnp.float32),
      ),
      mesh=mesh,
      scratch_types=[
          idx_t, idx_t,
          idx_t, idx_t,
          idx_t, idx_t,
          idx_t, idx_t,
          idx_t, idx_t,
          row_t, row_t,
          wrow_t, wrow_t,
          row_t, row_t,
          row_t, row_t,
          pltpu.VMEM_SHARED((N_PAD, DIM), jnp.float32),
          pltpu.VMEM_SHARED((CNT_ROWS, DIM), jnp.float32),
          pltpu.SemaphoreType.DMA, pltpu.SemaphoreType.DMA,
          pltpu.SemaphoreType.DMA, pltpu.SemaphoreType.DMA,
          pltpu.SemaphoreType.DMA, pltpu.SemaphoreType.DMA,
      ],
  )
  def body(h_hbm, wt_hbm, src_hbm, rel_hbm, dst_hbm, agg_hbm, cnt_hbm,
           src0, src1, rel0, rel1, dst0, dst1, sd0, sd1, sr0, sr1,
           x0, x1, w0, w1, m0, m1, o0, o1,
           agg_sh, cnt_sh,
           semi0, semi1, semg0, semg1, sems0, sems1):
    srcb = (src0, src1)
    relb = (rel0, rel1)
    dstb = (dst0, dst1)
    sdst = (sd0, sd1)
    sdr = (sr0, sr1)
    xb = (x0, x1)
    wb = (w0, w1)
    mb = (m0, m1)
    ob = (o0, o1)
    semi = (semi0, semi1)
    semg = (semg0, semg1)
    sems = (sems0, sems1)

    c = lax.axis_index("c")
    s = lax.axis_index("s")
    wid = c * NS + s

    lane = lax.iota(jnp.int32, LANES)
    zero16 = jnp.zeros((LANES,), jnp.float32)

    def zrow(e, carry):
      for j in range(DIM // LANES):
        m0[e, pl.ds(j * LANES, LANES)] = zero16
      return carry
    lax.fori_loop(0, CHUNK, zrow, None)

    tile_base = pl.multiple_of(s * ROWS_PER_TILE, 8)
    for k in range(ROWS_PER_TILE // CHUNK):
      pltpu.sync_copy(m0, agg_sh.at[pl.ds(tile_base + k * CHUNK, CHUNK)])

    @pl.when(s < CNT_TILES)
    def _zero_cnt():
      cbase = pl.multiple_of(s * 8, 8)
      pltpu.sync_copy(m0.at[pl.ds(0, 8)], cnt_sh.at[pl.ds(cbase, 8)])
    plsc.subcore_barrier()

    perm_base = (lane >> 2) * 4
    perms = [perm_base + i for i in range(BLK)]

    ebase = pl.multiple_of(wid * EW, 8)

    def stage_idx(ch, p):
      cb = pl.multiple_of(ebase + ch * CHUNK, 8)
      pltpu.async_copy(src_hbm.at[pl.ds(cb, CHUNK)], srcb[p], semi[p])
      pltpu.async_copy(rel_hbm.at[pl.ds(cb, CHUNK)], relb[p], semi[p])
      pltpu.async_copy(dst_hbm.at[pl.ds(cb, CHUNK)], dstb[p], semi[p])

    def wait_idx(p):
      for d in (srcb[p], relb[p], dstb[p]):
        pltpu.make_async_copy(src_hbm.at[pl.ds(0, CHUNK)], d, semi[p]).wait()

    def issue_gathers(p):
      pltpu.async_copy(h_hbm.at[srcb[p]], xb[p], semg[p])
      pltpu.async_copy(wt_hbm.at[relb[p]], wb[p], semg[p])

    def wait_gathers(p):
      pltpu.make_async_copy(h_hbm.at[srcb[p]], xb[p], semg[p]).wait()
      pltpu.make_async_copy(wt_hbm.at[relb[p]], wb[p], semg[p]).wait()

    def issue_scatters(p):
      pltpu.async_copy(mb[p], agg_sh.at[sdst[p]], sems[p], add=True)
      pltpu.async_copy(ob[p], cnt_sh.at[sdr[p]], sems[p], add=True)

    def wait_scatters(p):
      pltpu.make_async_copy(mb[p], agg_sh.at[sdst[p]], sems[p]).wait()
      pltpu.make_async_copy(ob[p], cnt_sh.at[sdr[p]], sems[p]).wait()

    def compute(p):
      dv = dstb[p][pl.ds(0, LANES)]
      sdst[p][pl.ds(0, LANES)] = dv
      sdr[p][pl.ds(0, LANES)] = lax.shift_right_logical(dv, 7)
      colg = lax.bitwise_and(dv, 127)
      for j in range(LANES):
        colv = jnp.broadcast_to(colg[j], (LANES,))
        for v in range(DIM // LANES):
          hit = (lane + v * LANES) == colv
          ob[p][j, pl.ds(LANES * v, LANES)] = jnp.where(hit, 1.0, 0.0)

      def edge_body(e, ecarry):
        for v in range(DIM // LANES):
          xv = xb[p][e, pl.ds(LANES * v, LANES)]
          acc = zero16
          for i in range(BLK):
            xs = jnp.take_along_axis(xv, perms[i], axis=0,
                                     mode="promise_in_bounds")
            wv = wb[p][e, pl.ds(i * DIM + LANES * v, LANES)]
            acc = acc + xs * wv
          mb[p][e, pl.ds(LANES * v, LANES)] = acc
        return ecarry
      lax.fori_loop(0, CHUNK, edge_body, None, unroll=2)

    stage_idx(0, 0)
    stage_idx(1, 1)
    wait_idx(0)
    issue_gathers(0)

    def outer(it, carry):
      for b in range(2):
        ch = it * 2 + b
        p, pn = b, 1 - b

        @pl.when(ch >= 2)
        def _drain():
          wait_scatters(p)

        @pl.when(ch < NCHUNK - 1)
        def _next_gather():
          wait_idx(pn)
          issue_gathers(pn)

        wait_gathers(p)
        compute(p)

        @pl.when(ch < NCHUNK - 2)
        def _next_idx():
          stage_idx(ch + 2, p)

        issue_scatters(p)
      return carry
    lax.fori_loop(0, NCHUNK // 2, outer, None)

    wait_scatters(0)
    wait_scatters(1)

    plsc.subcore_barrier()
    pltpu.sync_copy(agg_sh.at[pl.ds(tile_base, ROWS_PER_TILE)],
                    agg_hbm.at[c, pl.ds(tile_base, ROWS_PER_TILE)])

    @pl.when(s < CNT_TILES)
    def _write_cnt():
      cbase = pl.multiple_of(s * 8, 8)
      pltpu.sync_copy(cnt_sh.at[pl.ds(cbase, 8)],
                      cnt_hbm.at[c, pl.ds(cbase, 8)])

  return body(h, wt, src, rel, dst)


def _wt_permute_body(w_ref, p_ref, o_ref):
  o_ref[...] = jnp.dot(w_ref[...], p_ref[...],
                       preferred_element_type=jnp.float32)


def _wt_permute(b_weight):
  nr = b_weight.shape[0]
  w2 = b_weight.reshape(nr, 4 * DIM)
  col = jnp.arange(4 * DIM, dtype=jnp.int32)
  i_, b_, j_ = col // DIM, (col % DIM) // 4, col % 4
  perm = b_ * 16 + i_ * 4 + j_
  pmat = (jnp.arange(4 * DIM, dtype=jnp.int32)[:, None] ==
          perm[None, :]).astype(jnp.float32)
  blk_r = 400
  return pl.pallas_call(
      _wt_permute_body,
      grid=(nr // blk_r,),
      in_specs=[
          pl.BlockSpec((blk_r, 4 * DIM), lambda i: (i, 0)),
          pl.BlockSpec((4 * DIM, 4 * DIM), lambda i: (0, 0)),
      ],
      out_specs=pl.BlockSpec((blk_r, 4 * DIM), lambda i: (i, 0)),
      out_shape=jax.ShapeDtypeStruct((nr, 4 * DIM), jnp.float32),
  )(w2, pmat)


def _tc_combine_body(x_ref, w_ref, a_ref, c_ref, o_ref):
  msg = a_ref[0] + a_ref[1]
  cnt = c_ref[0] + c_ref[1]
  out = jnp.dot(x_ref[...], w_ref[...], preferred_element_type=jnp.float32)
  o_ref[...] = out + msg / jnp.maximum(cnt, 1.0)


def _tc_combine(h_pad, w_self, agg2, cnt2):
  blk_m = 1024
  grid = (N_PAD // blk_m,)
  return pl.pallas_call(
      _tc_combine_body,
      grid=grid,
      in_specs=[
          pl.BlockSpec((blk_m, DIM), lambda i: (i, 0)),
          pl.BlockSpec((DIM, DIM), lambda i: (0, 0)),
          pl.BlockSpec((NC, blk_m, DIM), lambda i: (0, i, 0)),
          pl.BlockSpec((NC, blk_m, 1), lambda i: (0, i, 0)),
      ],
      out_specs=pl.BlockSpec((blk_m, DIM), lambda i: (i, 0)),
      out_shape=jax.ShapeDtypeStruct((N_PAD, DIM), jnp.float32),
  )(h_pad, w_self, agg2, cnt2)


@jax.jit
def kernel(input_h, edges, b_weight, self_loop_weight):
  wt = _wt_permute(b_weight)
  src = edges[:, 0]
  rel = edges[:, 1]
  dst = edges[:, 2]
  pad = E_PAD - N_EDGES
  zpad = jnp.zeros((pad,), jnp.int32)
  src_p = jnp.concatenate([src, zpad])
  rel_p = jnp.concatenate([rel, zpad])
  dst_p = jnp.concatenate([dst, jnp.full((pad,), DUMMY_DST, jnp.int32)])
  agg2, cnt2 = _sc_aggregate(input_h, wt, src_p, rel_p, dst_p)
  cnt2 = cnt2.reshape(NC, N_PAD, 1)
  h_pad = jnp.concatenate(
      [input_h, jnp.zeros((N_PAD - N_NODES, DIM), jnp.float32)])
  out = _tc_combine(h_pad, self_loop_weight, agg2, cnt2)
  return out[:N_NODES]


# speedup vs baseline: 1.2267x; 1.2267x over previous
import functools

import jax
import jax.numpy as jnp
from jax import lax
from jax.experimental import pallas as pl
from jax.experimental.pallas import tpu as pltpu
from jax.experimental.pallas import tpu_sc as plsc

N_NODES = 10000
N_EDGES = 160000
DIM = 128
BLK = 4

NC = 2
NS = 16
NW = NC * NS
LANES = 16

N_PAD = 10240
DUMMY_DST = N_PAD - 1
CNT_ROWS = N_PAD // DIM
CHUNK = 16
NCHUNK = 316
EW = CHUNK * NCHUNK
E_PAD = EW * NW
ROWS_PER_TILE = N_PAD // NS
CNT_TILES = 10


def _sc_aggregate(h, wt, src, rel, dst):
  mesh = plsc.VectorSubcoreMesh(
      core_axis_name="c", subcore_axis_name="s", num_cores=NC, num_subcores=NS
  )

  idx_t = pltpu.VMEM((CHUNK,), jnp.int32)
  row_t = pltpu.VMEM((CHUNK, DIM), jnp.float32)
  wrow_t = pltpu.VMEM((CHUNK, 4 * DIM), jnp.float32)

  @functools.partial(
      pl.kernel,
      out_type=(
          jax.ShapeDtypeStruct((NC, N_PAD, DIM), jnp.float32),
          jax.ShapeDtypeStruct((NC, CNT_ROWS, DIM), jnp.float32),
      ),
      mesh=mesh,
      scratch_types=[
          idx_t, idx_t,
          idx_t, idx_t,
          idx_t, idx_t,
          idx_t, idx_t,
          idx_t, idx_t,
          idx_t, idx_t,
          row_t, row_t,
          wrow_t, wrow_t,
          row_t, row_t,
          row_t, row_t,
          pltpu.VMEM_SHARED((N_PAD, DIM), jnp.float32),
          pltpu.VMEM_SHARED((CNT_ROWS, DIM), jnp.float32),
          pltpu.SemaphoreType.DMA, pltpu.SemaphoreType.DMA,
          pltpu.SemaphoreType.DMA, pltpu.SemaphoreType.DMA,
          pltpu.SemaphoreType.DMA, pltpu.SemaphoreType.DMA,
          pltpu.SemaphoreType.DMA, pltpu.SemaphoreType.DMA,
      ],
  )
  def body(h_hbm, wt_hbm, ident_hbm, src_hbm, rel_hbm, dst_hbm,
           agg_hbm, cnt_hbm,
           src0, src1, rel0, rel1, dst0, dst1, sd0, sd1, sr0, sr1,
           sc0, sc1, x0, x1, w0, w1, m0, m1, o0, o1,
           agg_sh, cnt_sh,
           semi0, semi1, semg0, semg1, sems0, sems1, semo0, semo1):
    srcb = (src0, src1)
    relb = (rel0, rel1)
    dstb = (dst0, dst1)
    sdst = (sd0, sd1)
    sdr = (sr0, sr1)
    scol = (sc0, sc1)
    xb = (x0, x1)
    wb = (w0, w1)
    mb = (m0, m1)
    ob = (o0, o1)
    semi = (semi0, semi1)
    semg = (semg0, semg1)
    sems = (sems0, sems1)
    semo = (semo0, semo1)

    c = lax.axis_index("c")
    s = lax.axis_index("s")
    wid = c * NS + s

    lane = lax.iota(jnp.int32, LANES)
    zero16 = jnp.zeros((LANES,), jnp.float32)

    def zrow(e, carry):
      for j in range(DIM // LANES):
        m0[e, pl.ds(j * LANES, LANES)] = zero16
      return carry
    lax.fori_loop(0, CHUNK, zrow, None)

    tile_base = pl.multiple_of(s * ROWS_PER_TILE, 8)
    for k in range(ROWS_PER_TILE // CHUNK):
      pltpu.sync_copy(m0, agg_sh.at[pl.ds(tile_base + k * CHUNK, CHUNK)])

    @pl.when(s < CNT_TILES)
    def _zero_cnt():
      cbase = pl.multiple_of(s * 8, 8)
      pltpu.sync_copy(m0.at[pl.ds(0, 8)], cnt_sh.at[pl.ds(cbase, 8)])
    plsc.subcore_barrier()

    perm_base = (lane >> 2) * 4
    perms = [perm_base + i for i in range(BLK)]

    ebase = pl.multiple_of(wid * EW, 8)

    def stage_idx(ch, p):
      cb = pl.multiple_of(ebase + ch * CHUNK, 8)
      pltpu.async_copy(src_hbm.at[pl.ds(cb, CHUNK)], srcb[p], semi[p])
      pltpu.async_copy(rel_hbm.at[pl.ds(cb, CHUNK)], relb[p], semi[p])
      pltpu.async_copy(dst_hbm.at[pl.ds(cb, CHUNK)], dstb[p], semi[p])

    def wait_idx(p):
      for d in (srcb[p], relb[p], dstb[p]):
        pltpu.make_async_copy(src_hbm.at[pl.ds(0, CHUNK)], d, semi[p]).wait()

    def issue_gathers(p):
      pltpu.async_copy(h_hbm.at[srcb[p]], xb[p], semg[p])
      pltpu.async_copy(wt_hbm.at[relb[p]], wb[p], semg[p])

    def wait_gathers(p):
      pltpu.make_async_copy(h_hbm.at[srcb[p]], xb[p], semg[p]).wait()
      pltpu.make_async_copy(wt_hbm.at[relb[p]], wb[p], semg[p]).wait()

    def issue_scatters(p):
      pltpu.async_copy(mb[p], agg_sh.at[sdst[p]], sems[p], add=True)
      pltpu.async_copy(ob[p], cnt_sh.at[sdr[p]], sems[p], add=True)

    def wait_scatters(p):
      pltpu.make_async_copy(mb[p], agg_sh.at[sdst[p]], sems[p]).wait()
      pltpu.make_async_copy(ob[p], cnt_sh.at[sdr[p]], sems[p]).wait()

    def issue_ident_gather(p):
      dv = dstb[p][pl.ds(0, LANES)]
      scol[p][pl.ds(0, LANES)] = lax.bitwise_and(dv, 127)
      pltpu.async_copy(ident_hbm.at[scol[p]], ob[p], semo[p])

    def wait_ident_gather(p):
      pltpu.make_async_copy(ident_hbm.at[scol[p]], ob[p], semo[p]).wait()

    def compute(p):
      dv = dstb[p][pl.ds(0, LANES)]
      sdst[p][pl.ds(0, LANES)] = dv
      sdr[p][pl.ds(0, LANES)] = lax.shift_right_logical(dv, 7)

      def edge_body(e, ecarry):
        for v in range(DIM // LANES):
          xv = xb[p][e, pl.ds(LANES * v, LANES)]
          acc = zero16
          for i in range(BLK):
            xs = jnp.take_along_axis(xv, perms[i], axis=0,
                                     mode="promise_in_bounds")
            wv = wb[p][e, pl.ds(i * DIM + LANES * v, LANES)]
            acc = acc + xs * wv
          mb[p][e, pl.ds(LANES * v, LANES)] = acc
        return ecarry
      lax.fori_loop(0, CHUNK, edge_body, None)

    stage_idx(0, 0)
    stage_idx(1, 1)
    wait_idx(0)
    issue_gathers(0)

    def outer(it, carry):
      for b in range(2):
        ch = it * 2 + b
        p, pn = b, 1 - b

        @pl.when(ch >= 2)
        def _drain():
          wait_scatters(p)

        issue_ident_gather(p)

        @pl.when(ch < NCHUNK - 1)
        def _next_gather():
          wait_idx(pn)
          issue_gathers(pn)

        wait_gathers(p)
        compute(p)

        @pl.when(ch < NCHUNK - 2)
        def _next_idx():
          stage_idx(ch + 2, p)

        wait_ident_gather(p)
        issue_scatters(p)
      return carry
    lax.fori_loop(0, NCHUNK // 2, outer, None)

    wait_scatters(0)
    wait_scatters(1)

    plsc.subcore_barrier()
    pltpu.sync_copy(agg_sh.at[pl.ds(tile_base, ROWS_PER_TILE)],
                    agg_hbm.at[c, pl.ds(tile_base, ROWS_PER_TILE)])

    @pl.when(s < CNT_TILES)
    def _write_cnt():
      cbase = pl.multiple_of(s * 8, 8)
      pltpu.sync_copy(cnt_sh.at[pl.ds(cbase, 8)],
                      cnt_hbm.at[c, pl.ds(cbase, 8)])

  ident = jnp.eye(DIM, dtype=jnp.float32)
  return body(h, wt, ident, src, rel, dst)


def _wt_permute_body(w_ref, p_ref, o_ref):
  o_ref[...] = jnp.dot(w_ref[...], p_ref[...],
                       preferred_element_type=jnp.float32)


def _wt_permute(b_weight):
  nr = b_weight.shape[0]
  w2 = b_weight.reshape(nr, 4 * DIM)
  col = jnp.arange(4 * DIM, dtype=jnp.int32)
  i_, b_, j_ = col // DIM, (col % DIM) // 4, col % 4
  perm = b_ * 16 + i_ * 4 + j_
  pmat = (jnp.arange(4 * DIM, dtype=jnp.int32)[:, None] ==
          perm[None, :]).astype(jnp.float32)
  blk_r = 400
  return pl.pallas_call(
      _wt_permute_body,
      grid=(nr // blk_r,),
      in_specs=[
          pl.BlockSpec((blk_r, 4 * DIM), lambda i: (i, 0)),
          pl.BlockSpec((4 * DIM, 4 * DIM), lambda i: (0, 0)),
      ],
      out_specs=pl.BlockSpec((blk_r, 4 * DIM), lambda i: (i, 0)),
      out_shape=jax.ShapeDtypeStruct((nr, 4 * DIM), jnp.float32),
  )(w2, pmat)


def _tc_combine_body(x_ref, w_ref, a_ref, c_ref, o_ref):
  msg = a_ref[0] + a_ref[1]
  cnt = c_ref[0] + c_ref[1]
  out = jnp.dot(x_ref[...], w_ref[...], preferred_element_type=jnp.float32)
  o_ref[...] = out + msg / jnp.maximum(cnt, 1.0)


def _tc_combine(h_pad, w_self, agg2, cnt2):
  blk_m = 1024
  grid = (N_PAD // blk_m,)
  return pl.pallas_call(
      _tc_combine_body,
      grid=grid,
      in_specs=[
          pl.BlockSpec((blk_m, DIM), lambda i: (i, 0)),
          pl.BlockSpec((DIM, DIM), lambda i: (0, 0)),
          pl.BlockSpec((NC, blk_m, DIM), lambda i: (0, i, 0)),
          pl.BlockSpec((NC, blk_m, 1), lambda i: (0, i, 0)),
      ],
      out_specs=pl.BlockSpec((blk_m, DIM), lambda i: (i, 0)),
      out_shape=jax.ShapeDtypeStruct((N_PAD, DIM), jnp.float32),
  )(h_pad, w_self, agg2, cnt2)


@jax.jit
def kernel(input_h, edges, b_weight, self_loop_weight):
  wt = _wt_permute(b_weight)
  src = edges[:, 0]
  rel = edges[:, 1]
  dst = edges[:, 2]
  pad = E_PAD - N_EDGES
  zpad = jnp.zeros((pad,), jnp.int32)
  src_p = jnp.concatenate([src, zpad])
  rel_p = jnp.concatenate([rel, zpad])
  dst_p = jnp.concatenate([dst, jnp.full((pad,), DUMMY_DST, jnp.int32)])
  agg2, cnt2 = _sc_aggregate(input_h, wt, src_p, rel_p, dst_p)
  cnt2 = cnt2.reshape(NC, N_PAD, 1)
  h_pad = jnp.concatenate(
      [input_h, jnp.zeros((N_PAD - N_NODES, DIM), jnp.float32)])
  out = _tc_combine(h_pad, self_loop_weight, agg2, cnt2)
  return out[:N_NODES]


# speedup vs baseline: 1.6220x; 1.3223x over previous
import functools

import jax
import jax.numpy as jnp
from jax import lax
from jax.experimental import pallas as pl
from jax.experimental.pallas import tpu as pltpu
from jax.experimental.pallas import tpu_sc as plsc

N_NODES = 10000
N_EDGES = 160000
DIM = 128
BLK = 4

NC = 2
NS = 16
NW = NC * NS
LANES = 16

N_PAD = 10240
DUMMY_DST = N_PAD - 1
CNT_ROWS = N_PAD // DIM
CHUNK = 16
NCHUNK = 316
EW = CHUNK * NCHUNK
E_PAD = EW * NW
ROWS_PER_TILE = N_PAD // NS
CNT_TILES = 10


def _sc_aggregate(h, wt, src, rel, dst):
  mesh = plsc.VectorSubcoreMesh(
      core_axis_name="c", subcore_axis_name="s", num_cores=NC, num_subcores=NS
  )

  idx_t = pltpu.VMEM((CHUNK,), jnp.int32)
  row_t = pltpu.VMEM((CHUNK, DIM), jnp.float32)
  wrow_t = pltpu.VMEM((CHUNK, 4 * DIM), jnp.float32)

  @functools.partial(
      pl.kernel,
      out_type=(
          jax.ShapeDtypeStruct((NC, N_PAD, DIM), jnp.float32),
          jax.ShapeDtypeStruct((NC, CNT_ROWS, DIM), jnp.float32),
      ),
      mesh=mesh,
      scratch_types=[
          idx_t, idx_t,
          idx_t, idx_t,
          idx_t, idx_t,
          idx_t, idx_t,
          idx_t, idx_t,
          idx_t, idx_t,
          row_t, row_t,
          wrow_t, wrow_t,
          row_t, row_t,
          row_t, row_t,
          pltpu.VMEM_SHARED((N_PAD, DIM), jnp.float32),
          pltpu.VMEM_SHARED((CNT_ROWS, DIM), jnp.float32),
          pltpu.SemaphoreType.DMA, pltpu.SemaphoreType.DMA,
          pltpu.SemaphoreType.DMA, pltpu.SemaphoreType.DMA,
          pltpu.SemaphoreType.DMA, pltpu.SemaphoreType.DMA,
          pltpu.SemaphoreType.DMA, pltpu.SemaphoreType.DMA,
      ],
  )
  def body(h_hbm, wt_hbm, ident_hbm, src_hbm, rel_hbm, dst_hbm,
           agg_hbm, cnt_hbm,
           src0, src1, rel0, rel1, dst0, dst1, sd0, sd1, sr0, sr1,
           sc0, sc1, x0, x1, w0, w1, m0, m1, o0, o1,
           agg_sh, cnt_sh,
           semi0, semi1, semg0, semg1, sems0, sems1, semo0, semo1):
    srcb = (src0, src1)
    relb = (rel0, rel1)
    dstb = (dst0, dst1)
    sdst = (sd0, sd1)
    sdr = (sr0, sr1)
    scol = (sc0, sc1)
    xb = (x0, x1)
    wb = (w0, w1)
    mb = (m0, m1)
    ob = (o0, o1)
    semi = (semi0, semi1)
    semg = (semg0, semg1)
    sems = (sems0, sems1)
    semo = (semo0, semo1)

    c = lax.axis_index("c")
    s = lax.axis_index("s")
    wid = c * NS + s

    lane = lax.iota(jnp.int32, LANES)
    zero16 = jnp.zeros((LANES,), jnp.float32)

    def zrow(e, carry):
      for j in range(DIM // LANES):
        m0[e, pl.ds(j * LANES, LANES)] = zero16
      return carry
    lax.fori_loop(0, CHUNK, zrow, None)

    tile_base = pl.multiple_of(s * ROWS_PER_TILE, 8)
    for k in range(ROWS_PER_TILE // CHUNK):
      pltpu.sync_copy(m0, agg_sh.at[pl.ds(tile_base + k * CHUNK, CHUNK)])

    @pl.when(s < CNT_TILES)
    def _zero_cnt():
      cbase = pl.multiple_of(s * 8, 8)
      pltpu.sync_copy(m0.at[pl.ds(0, 8)], cnt_sh.at[pl.ds(cbase, 8)])
    plsc.subcore_barrier()

    perm_base = (lane >> 2) * 4
    perms = [perm_base + i for i in range(BLK)]

    ebase = pl.multiple_of(wid * EW, 8)

    def stage_idx(ch, p):
      cb = pl.multiple_of(ebase + ch * CHUNK, 8)
      pltpu.async_copy(src_hbm.at[pl.ds(cb, CHUNK)], srcb[p], semi[p])
      pltpu.async_copy(rel_hbm.at[pl.ds(cb, CHUNK)], relb[p], semi[p])
      pltpu.async_copy(dst_hbm.at[pl.ds(cb, CHUNK)], dstb[p], semi[p])

    def wait_idx(p):
      for d in (srcb[p], relb[p], dstb[p]):
        pltpu.make_async_copy(src_hbm.at[pl.ds(0, CHUNK)], d, semi[p]).wait()

    def issue_gathers(p):
      pltpu.async_copy(h_hbm.at[srcb[p]], xb[p], semg[p])
      pltpu.async_copy(wt_hbm.at[relb[p]], wb[p], semg[p])

    def wait_gathers(p):
      pltpu.make_async_copy(h_hbm.at[srcb[p]], xb[p], semg[p]).wait()
      pltpu.make_async_copy(wt_hbm.at[relb[p]], wb[p], semg[p]).wait()

    def issue_scatters(p):
      pltpu.async_copy(mb[p], agg_sh.at[sdst[p]], sems[p], add=True)
      pltpu.async_copy(ob[p], cnt_sh.at[sdr[p]], sems[p], add=True)

    def wait_scatters(p):
      pltpu.make_async_copy(mb[p], agg_sh.at[sdst[p]], sems[p]).wait()
      pltpu.make_async_copy(ob[p], cnt_sh.at[sdr[p]], sems[p]).wait()

    def issue_ident_gather(p):
      dv = dstb[p][pl.ds(0, LANES)]
      scol[p][pl.ds(0, LANES)] = lax.bitwise_and(dv, 127)
      pltpu.async_copy(ident_hbm.at[scol[p]], ob[p], semo[p])

    def wait_ident_gather(p):
      pltpu.make_async_copy(ident_hbm.at[scol[p]], ob[p], semo[p]).wait()

    def compute(p):
      dv = dstb[p][pl.ds(0, LANES)]
      sdst[p][pl.ds(0, LANES)] = dv
      sdr[p][pl.ds(0, LANES)] = lax.shift_right_logical(dv, 7)

      @plsc.parallel_loop(0, CHUNK)
      def edge_body(e):
        for v in range(DIM // LANES):
          xv = xb[p][e, pl.ds(LANES * v, LANES)]
          acc = zero16
          for i in range(BLK):
            xs = jnp.take_along_axis(xv, perms[i], axis=0,
                                     mode="promise_in_bounds")
            wv = wb[p][e, pl.ds(i * DIM + LANES * v, LANES)]
            acc = acc + xs * wv
          mb[p][e, pl.ds(LANES * v, LANES)] = acc

    stage_idx(0, 0)
    stage_idx(1, 1)
    wait_idx(0)
    issue_gathers(0)

    def outer(it, carry):
      for b in range(2):
        ch = it * 2 + b
        p, pn = b, 1 - b

        @pl.when(ch >= 2)
        def _drain():
          wait_scatters(p)

        issue_ident_gather(p)

        @pl.when(ch < NCHUNK - 1)
        def _next_gather():
          wait_idx(pn)
          issue_gathers(pn)

        wait_gathers(p)
        compute(p)

        @pl.when(ch < NCHUNK - 2)
        def _next_idx():
          stage_idx(ch + 2, p)

        wait_ident_gather(p)
        issue_scatters(p)
      return carry
    lax.fori_loop(0, NCHUNK // 2, outer, None)

    wait_scatters(0)
    wait_scatters(1)

    plsc.subcore_barrier()
    pltpu.sync_copy(agg_sh.at[pl.ds(tile_base, ROWS_PER_TILE)],
                    agg_hbm.at[c, pl.ds(tile_base, ROWS_PER_TILE)])

    @pl.when(s < CNT_TILES)
    def _write_cnt():
      cbase = pl.multiple_of(s * 8, 8)
      pltpu.sync_copy(cnt_sh.at[pl.ds(cbase, 8)],
                      cnt_hbm.at[c, pl.ds(cbase, 8)])

  ident = jnp.eye(DIM, dtype=jnp.float32)
  return body(h, wt, ident, src, rel, dst)


def _wt_permute_body(w_ref, p_ref, o_ref):
  o_ref[...] = jnp.dot(w_ref[...], p_ref[...],
                       preferred_element_type=jnp.float32)


def _wt_permute(b_weight):
  nr = b_weight.shape[0]
  w2 = b_weight.reshape(nr, 4 * DIM)
  col = jnp.arange(4 * DIM, dtype=jnp.int32)
  i_, b_, j_ = col // DIM, (col % DIM) // 4, col % 4
  perm = b_ * 16 + i_ * 4 + j_
  pmat = (jnp.arange(4 * DIM, dtype=jnp.int32)[:, None] ==
          perm[None, :]).astype(jnp.float32)
  blk_r = 400
  return pl.pallas_call(
      _wt_permute_body,
      grid=(nr // blk_r,),
      in_specs=[
          pl.BlockSpec((blk_r, 4 * DIM), lambda i: (i, 0)),
          pl.BlockSpec((4 * DIM, 4 * DIM), lambda i: (0, 0)),
      ],
      out_specs=pl.BlockSpec((blk_r, 4 * DIM), lambda i: (i, 0)),
      out_shape=jax.ShapeDtypeStruct((nr, 4 * DIM), jnp.float32),
  )(w2, pmat)


def _tc_combine_body(x_ref, w_ref, a_ref, c_ref, o_ref):
  msg = a_ref[0] + a_ref[1]
  cnt = c_ref[0] + c_ref[1]
  out = jnp.dot(x_ref[...], w_ref[...], preferred_element_type=jnp.float32)
  o_ref[...] = out + msg / jnp.maximum(cnt, 1.0)


def _tc_combine(h_pad, w_self, agg2, cnt2):
  blk_m = 1024
  grid = (N_PAD // blk_m,)
  return pl.pallas_call(
      _tc_combine_body,
      grid=grid,
      in_specs=[
          pl.BlockSpec((blk_m, DIM), lambda i: (i, 0)),
          pl.BlockSpec((DIM, DIM), lambda i: (0, 0)),
          pl.BlockSpec((NC, blk_m, DIM), lambda i: (0, i, 0)),
          pl.BlockSpec((NC, blk_m, 1), lambda i: (0, i, 0)),
      ],
      out_specs=pl.BlockSpec((blk_m, DIM), lambda i: (i, 0)),
      out_shape=jax.ShapeDtypeStruct((N_PAD, DIM), jnp.float32),
  )(h_pad, w_self, agg2, cnt2)


@jax.jit
def kernel(input_h, edges, b_weight, self_loop_weight):
  wt = _wt_permute(b_weight)
  src = edges[:, 0]
  rel = edges[:, 1]
  dst = edges[:, 2]
  pad = E_PAD - N_EDGES
  zpad = jnp.zeros((pad,), jnp.int32)
  src_p = jnp.concatenate([src, zpad])
  rel_p = jnp.concatenate([rel, zpad])
  dst_p = jnp.concatenate([dst, jnp.full((pad,), DUMMY_DST, jnp.int32)])
  agg2, cnt2 = _sc_aggregate(input_h, wt, src_p, rel_p, dst_p)
  cnt2 = cnt2.reshape(NC, N_PAD, 1)
  h_pad = jnp.concatenate(
      [input_h, jnp.zeros((N_PAD - N_NODES, DIM), jnp.float32)])
  out = _tc_combine(h_pad, self_loop_weight, agg2, cnt2)
  return out[:N_NODES]


# speedup vs baseline: 1.6244x; 1.0015x over previous
import functools

import jax
import jax.numpy as jnp
from jax import lax
from jax.experimental import pallas as pl
from jax.experimental.pallas import tpu as pltpu
from jax.experimental.pallas import tpu_sc as plsc

N_NODES = 10000
N_EDGES = 160000
DIM = 128
BLK = 4

NC = 2
NS = 16
NW = NC * NS
LANES = 16

N_PAD = 10240
DUMMY_DST = N_PAD - 1
CNT_ROWS = N_PAD // DIM
CHUNK = 16
NCHUNK = 316
EW = CHUNK * NCHUNK
E_PAD = EW * NW
ROWS_PER_TILE = N_PAD // NS
CNT_TILES = 10


def _sc_aggregate(h, wt, src, rel, dst):
  mesh = plsc.VectorSubcoreMesh(
      core_axis_name="c", subcore_axis_name="s", num_cores=NC, num_subcores=NS
  )

  idx_t = pltpu.VMEM((CHUNK,), jnp.int32)
  row_t = pltpu.VMEM((CHUNK, DIM), jnp.float32)
  wrow_t = pltpu.VMEM((CHUNK, 4 * DIM), jnp.float32)

  @functools.partial(
      pl.kernel,
      out_type=(
          jax.ShapeDtypeStruct((NC, N_PAD, DIM), jnp.float32),
          jax.ShapeDtypeStruct((NC, CNT_ROWS, DIM), jnp.float32),
      ),
      mesh=mesh,
      scratch_types=[
          idx_t, idx_t,
          idx_t, idx_t,
          idx_t, idx_t,
          idx_t, idx_t,
          idx_t, idx_t,
          idx_t, idx_t,
          row_t, row_t,
          wrow_t, wrow_t,
          row_t, row_t,
          row_t, row_t,
          pltpu.VMEM_SHARED((N_PAD, DIM), jnp.float32),
          pltpu.VMEM_SHARED((CNT_ROWS, DIM), jnp.float32),
          pltpu.SemaphoreType.DMA, pltpu.SemaphoreType.DMA,
          pltpu.SemaphoreType.DMA, pltpu.SemaphoreType.DMA,
          pltpu.SemaphoreType.DMA, pltpu.SemaphoreType.DMA,
          pltpu.SemaphoreType.DMA, pltpu.SemaphoreType.DMA,
      ],
  )
  def body(h_hbm, wt_hbm, ident_hbm, src_hbm, rel_hbm, dst_hbm,
           agg_hbm, cnt_hbm,
           src0, src1, rel0, rel1, dst0, dst1, sd0, sd1, sr0, sr1,
           sc0, sc1, x0, x1, w0, w1, m0, m1, o0, o1,
           agg_sh, cnt_sh,
           semi0, semi1, semg0, semg1, sems0, sems1, semo0, semo1):
    srcb = (src0, src1)
    relb = (rel0, rel1)
    dstb = (dst0, dst1)
    sdst = (sd0, sd1)
    sdr = (sr0, sr1)
    scol = (sc0, sc1)
    xb = (x0, x1)
    wb = (w0, w1)
    mb = (m0, m1)
    ob = (o0, o1)
    semi = (semi0, semi1)
    semg = (semg0, semg1)
    sems = (sems0, sems1)
    semo = (semo0, semo1)

    c = lax.axis_index("c")
    s = lax.axis_index("s")
    wid = c * NS + s

    lane = lax.iota(jnp.int32, LANES)
    zero16 = jnp.zeros((LANES,), jnp.float32)

    def zrow(e, carry):
      for j in range(DIM // LANES):
        m0[e, pl.ds(j * LANES, LANES)] = zero16
      return carry
    lax.fori_loop(0, CHUNK, zrow, None)

    tile_base = pl.multiple_of(s * ROWS_PER_TILE, 8)
    for k in range(ROWS_PER_TILE // CHUNK):
      pltpu.sync_copy(m0, agg_sh.at[pl.ds(tile_base + k * CHUNK, CHUNK)])

    @pl.when(s < CNT_TILES)
    def _zero_cnt():
      cbase = pl.multiple_of(s * 8, 8)
      pltpu.sync_copy(m0.at[pl.ds(0, 8)], cnt_sh.at[pl.ds(cbase, 8)])
    plsc.subcore_barrier()

    perm_base = (lane >> 2) * 4
    perms = [perm_base + i for i in range(BLK)]

    ebase = pl.multiple_of(wid * EW, 8)

    def stage_idx(ch, p):
      cb = pl.multiple_of(ebase + ch * CHUNK, 8)
      pltpu.async_copy(src_hbm.at[pl.ds(cb, CHUNK)], srcb[p], semi[p])
      pltpu.async_copy(rel_hbm.at[pl.ds(cb, CHUNK)], relb[p], semi[p])
      pltpu.async_copy(dst_hbm.at[pl.ds(cb, CHUNK)], dstb[p], semi[p])

    def wait_idx(p):
      for d in (srcb[p], relb[p], dstb[p]):
        pltpu.make_async_copy(src_hbm.at[pl.ds(0, CHUNK)], d, semi[p]).wait()

    def issue_gathers(p):
      pltpu.async_copy(h_hbm.at[srcb[p]], xb[p], semg[p])
      pltpu.async_copy(wt_hbm.at[relb[p]], wb[p], semg[p])

    def wait_gathers(p):
      pltpu.make_async_copy(h_hbm.at[srcb[p]], xb[p], semg[p]).wait()
      pltpu.make_async_copy(wt_hbm.at[relb[p]], wb[p], semg[p]).wait()

    def issue_scatters(p):
      pltpu.async_copy(mb[p], agg_sh.at[sdst[p]], sems[p], add=True)
      pltpu.async_copy(ob[p], cnt_sh.at[sdr[p]], sems[p], add=True)

    def wait_scatters(p):
      pltpu.make_async_copy(mb[p], agg_sh.at[sdst[p]], sems[p]).wait()
      pltpu.make_async_copy(ob[p], cnt_sh.at[sdr[p]], sems[p]).wait()

    def issue_ident_gather(p):
      dv = dstb[p][pl.ds(0, LANES)]
      scol[p][pl.ds(0, LANES)] = lax.bitwise_and(dv, 127)
      pltpu.async_copy(ident_hbm.at[scol[p]], ob[p], semo[p])

    def wait_ident_gather(p):
      pltpu.make_async_copy(ident_hbm.at[scol[p]], ob[p], semo[p]).wait()

    def compute(p):
      dv = dstb[p][pl.ds(0, LANES)]
      sdst[p][pl.ds(0, LANES)] = dv
      sdr[p][pl.ds(0, LANES)] = lax.shift_right_logical(dv, 7)

      @plsc.parallel_loop(0, CHUNK, unroll=2)
      def edge_body(e):
        for v in range(DIM // LANES):
          xv = xb[p][e, pl.ds(LANES * v, LANES)]
          acc = zero16
          for i in range(BLK):
            xs = jnp.take_along_axis(xv, perms[i], axis=0,
                                     mode="promise_in_bounds")
            wv = wb[p][e, pl.ds(i * DIM + LANES * v, LANES)]
            acc = acc + xs * wv
          mb[p][e, pl.ds(LANES * v, LANES)] = acc

    stage_idx(0, 0)
    stage_idx(1, 1)
    wait_idx(0)
    issue_gathers(0)

    def outer(it, carry):
      for b in range(2):
        ch = it * 2 + b
        p, pn = b, 1 - b

        @pl.when(ch >= 2)
        def _drain():
          wait_scatters(p)

        issue_ident_gather(p)

        @pl.when(ch < NCHUNK - 1)
        def _next_gather():
          wait_idx(pn)
          issue_gathers(pn)

        wait_gathers(p)
        compute(p)

        @pl.when(ch < NCHUNK - 2)
        def _next_idx():
          stage_idx(ch + 2, p)

        wait_ident_gather(p)
        issue_scatters(p)
      return carry
    lax.fori_loop(0, NCHUNK // 2, outer, None)

    wait_scatters(0)
    wait_scatters(1)

    plsc.subcore_barrier()
    pltpu.sync_copy(agg_sh.at[pl.ds(tile_base, ROWS_PER_TILE)],
                    agg_hbm.at[c, pl.ds(tile_base, ROWS_PER_TILE)])

    @pl.when(s < CNT_TILES)
    def _write_cnt():
      cbase = pl.multiple_of(s * 8, 8)
      pltpu.sync_copy(cnt_sh.at[pl.ds(cbase, 8)],
                      cnt_hbm.at[c, pl.ds(cbase, 8)])

  ident = jnp.eye(DIM, dtype=jnp.float32)
  return body(h, wt, ident, src, rel, dst)


def _wt_permute_body(w_ref, p_ref, o_ref):
  o_ref[...] = jnp.dot(w_ref[...], p_ref[...],
                       preferred_element_type=jnp.float32)


def _wt_permute(b_weight):
  nr = b_weight.shape[0]
  w2 = b_weight.reshape(nr, 4 * DIM)
  col = jnp.arange(4 * DIM, dtype=jnp.int32)
  i_, b_, j_ = col // DIM, (col % DIM) // 4, col % 4
  perm = b_ * 16 + i_ * 4 + j_
  pmat = (jnp.arange(4 * DIM, dtype=jnp.int32)[:, None] ==
          perm[None, :]).astype(jnp.float32)
  blk_r = 400
  return pl.pallas_call(
      _wt_permute_body,
      grid=(nr // blk_r,),
      in_specs=[
          pl.BlockSpec((blk_r, 4 * DIM), lambda i: (i, 0)),
          pl.BlockSpec((4 * DIM, 4 * DIM), lambda i: (0, 0)),
      ],
      out_specs=pl.BlockSpec((blk_r, 4 * DIM), lambda i: (i, 0)),
      out_shape=jax.ShapeDtypeStruct((nr, 4 * DIM), jnp.float32),
  )(w2, pmat)


def _tc_combine_body(x_ref, w_ref, a_ref, c_ref, o_ref):
  msg = a_ref[0] + a_ref[1]
  cnt = c_ref[0] + c_ref[1]
  out = jnp.dot(x_ref[...], w_ref[...], preferred_element_type=jnp.float32)
  o_ref[...] = out + msg / jnp.maximum(cnt, 1.0)


def _tc_combine(h_pad, w_self, agg2, cnt2):
  blk_m = 1024
  grid = (N_PAD // blk_m,)
  return pl.pallas_call(
      _tc_combine_body,
      grid=grid,
      in_specs=[
          pl.BlockSpec((blk_m, DIM), lambda i: (i, 0)),
          pl.BlockSpec((DIM, DIM), lambda i: (0, 0)),
          pl.BlockSpec((NC, blk_m, DIM), lambda i: (0, i, 0)),
          pl.BlockSpec((NC, blk_m, 1), lambda i: (0, i, 0)),
      ],
      out_specs=pl.BlockSpec((blk_m, DIM), lambda i: (i, 0)),
      out_shape=jax.ShapeDtypeStruct((N_PAD, DIM), jnp.float32),
  )(h_pad, w_self, agg2, cnt2)


@jax.jit
def kernel(input_h, edges, b_weight, self_loop_weight):
  wt = _wt_permute(b_weight)
  src = edges[:, 0]
  rel = edges[:, 1]
  dst = edges[:, 2]
  pad = E_PAD - N_EDGES
  zpad = jnp.zeros((pad,), jnp.int32)
  src_p = jnp.concatenate([src, zpad])
  rel_p = jnp.concatenate([rel, zpad])
  dst_p = jnp.concatenate([dst, jnp.full((pad,), DUMMY_DST, jnp.int32)])
  agg2, cnt2 = _sc_aggregate(input_h, wt, src_p, rel_p, dst_p)
  cnt2 = cnt2.reshape(NC, N_PAD, 1)
  h_pad = jnp.concatenate(
      [input_h, jnp.zeros((N_PAD - N_NODES, DIM), jnp.float32)])
  out = _tc_combine(h_pad, self_loop_weight, agg2, cnt2)
  return out[:N_NODES]
